# Initial kernel scaffold; baseline (speedup 1.0000x reference)
import jax
import jax.numpy as jnp
from jax.experimental import pallas as pl


def kernel(byte_stream, target_geodesic, emb_table, proj_W, proj_b):
    raise NotImplementedError("write your pallas kernel here")


# speedup vs baseline: 1.8480x; 1.8480x over previous
import functools

import jax
import jax.numpy as jnp
from jax import lax
from jax.experimental import pallas as pl
from jax.experimental.pallas import tpu as pltpu
from jax.experimental.pallas import tpu_sc as plsc


def _table_body(emb_ref, wt_ref, b_ref, out_ref):
    q = jnp.dot(emb_ref[...], wt_ref[...], preferred_element_type=jnp.float32)
    q = q + b_ref[...]
    n = jnp.sqrt(jnp.sum(q * q, axis=-1, keepdims=True))
    out_ref[...] = q / (n + 1e-12)


def _loss_body(n_total, part_ref, out_ref):
    s = jnp.sum(part_ref[...])
    out_ref[...] = jnp.reshape(1.0 - s * (1.0 / n_total), (1, 1))


@functools.lru_cache(maxsize=None)
def _make_sc_kernel(n_total, chunk):
    info = plsc.get_sparse_core_info()
    nc, ns = info.num_cores, info.num_subcores
    nw = nc * ns
    per_w = n_total // nw
    n_chunks = per_w // chunk
    unroll = 4
    n_inner = chunk // (16 * unroll)
    mesh = plsc.VectorSubcoreMesh(core_axis_name="c", subcore_axis_name="s")

    @functools.partial(
        pl.kernel,
        mesh=mesh,
        compiler_params=pltpu.CompilerParams(needs_layout_passes=False),
        out_type=(
            jax.ShapeDtypeStruct((n_total * 4,), jnp.float32),
            jax.ShapeDtypeStruct((nw, 16), jnp.float32),
        ),
        scratch_types=[
            pltpu.VMEM((1024,), jnp.float32),
            pltpu.VMEM((chunk,), jnp.int32),
            pltpu.VMEM((chunk * 4,), jnp.float32),
            pltpu.VMEM((chunk * 4,), jnp.float32),
            pltpu.VMEM((16,), jnp.float32),
        ],
    )
    def sc_kernel(bytes_hbm, tgt_hbm, ct_hbm, out_hbm, part_hbm,
                  table_v, idx_v, tgt_v, rows_v, acc_v):
        wid = lax.axis_index("s") * nc + lax.axis_index("c")
        base = wid * per_w
        pltpu.sync_copy(ct_hbm, table_v)
        iota = lax.iota(jnp.int32, 16)
        fvecs = [iota * 4 + k for k in range(4)]

        def chunk_body(g, acc):
            off = base + g * chunk
            pltpu.sync_copy(bytes_hbm.at[pl.ds(off, chunk)], idx_v)
            pltpu.sync_copy(tgt_hbm.at[pl.ds(off * 4, chunk * 4)], tgt_v)

            def inner(i, acc):
                for j in range(unroll):
                    s0 = (i * unroll + j) * 16
                    b16 = idx_v[pl.ds(s0, 16)]
                    b4 = b16 * 4
                    d = None
                    for k in range(4):
                        fk = fvecs[k] + (s0 * 4)
                        ck = plsc.load_gather(table_v, [b4 + k])
                        tk = plsc.load_gather(tgt_v, [fk])
                        plsc.store_scatter(rows_v, [fk], ck)
                        d = ck * tk if d is None else d + ck * tk
                    acc = acc + d * d
                return acc

            acc = lax.fori_loop(0, n_inner, inner, acc)
            pltpu.sync_copy(rows_v, out_hbm.at[pl.ds(off * 4, chunk * 4)])
            return acc

        acc = lax.fori_loop(0, n_chunks, chunk_body,
                            jnp.zeros((16,), jnp.float32))
        acc_v[...] = acc
        pltpu.sync_copy(acc_v, part_hbm.at[wid])

    return sc_kernel


def kernel(byte_stream, target_geodesic, emb_table, proj_W, proj_b):
    b, s = byte_stream.shape
    n_total = b * s
    bytes_flat = byte_stream.reshape(n_total)
    tgt_flat = target_geodesic.reshape(n_total * 4)

    code_table = pl.pallas_call(
        _table_body,
        out_shape=jax.ShapeDtypeStruct((256, 4), jnp.float32),
    )(emb_table, proj_W.T, proj_b.reshape(1, 4))

    sc = _make_sc_kernel(n_total, 6400)
    code_flat, partials = sc(bytes_flat, tgt_flat, code_table.reshape(1024))

    loss2d = pl.pallas_call(
        functools.partial(_loss_body, float(n_total)),
        out_shape=jax.ShapeDtypeStruct((1, 1), jnp.float32),
    )(partials)
    loss = loss2d.reshape(())

    code_path = code_flat.reshape(b, s, 4)
    return (loss, code_path, target_geodesic, jnp.float32(0.1))


# speedup vs baseline: 111.5254x; 60.3494x over previous
import functools

import jax
import jax.numpy as jnp
from jax import lax
from jax.experimental import pallas as pl
from jax.experimental.pallas import tpu as pltpu
from jax.experimental.pallas import tpu_sc as plsc


def _loss_body(n_total, part_ref, out_ref):
    s = jnp.sum(part_ref[...])
    out_ref[...] = jnp.reshape(1.0 - s * (1.0 / n_total), (1, 1))


def _rsqrt16(x):
    i = plsc.bitcast(x, jnp.int32)
    y = plsc.bitcast(jnp.int32(0x5F3759DF) - (i >> 1), jnp.float32)
    half = x * 0.5
    for _ in range(3):
        y = y * (1.5 - half * y * y)
    return y


@functools.lru_cache(maxsize=None)
def _make_sc_kernel(s_dim, b_dim):
    info = plsc.get_sparse_core_info()
    nc, ns = info.num_cores, info.num_subcores
    nw = nc * ns
    bw = b_dim // nw
    n_st = s_dim // 8
    unroll = 8
    groups = 8 * (bw // 16)
    cpr = bw // 16
    mesh = plsc.VectorSubcoreMesh(core_axis_name="c", subcore_axis_name="s")

    @functools.partial(
        pl.kernel,
        mesh=mesh,
        compiler_params=pltpu.CompilerParams(needs_layout_passes=False),
        out_type=(
            jax.ShapeDtypeStruct((s_dim, 4, b_dim), jnp.float32),
            jax.ShapeDtypeStruct((nw, 16), jnp.float32),
        ),
        scratch_types=[
            pltpu.VMEM((1024,), jnp.float32),
            pltpu.VMEM((4, 256), jnp.float32),
            pltpu.VMEM((24,), jnp.float32),
            pltpu.VMEM((2, 8, bw), jnp.int32),
            pltpu.VMEM((2, 8, 4, bw), jnp.float32),
            pltpu.VMEM((2, 8, 4, bw), jnp.float32),
            pltpu.VMEM((16,), jnp.float32),
            pltpu.SemaphoreType.DMA,
            pltpu.SemaphoreType.DMA,
            pltpu.SemaphoreType.DMA,
            pltpu.SemaphoreType.DMA,
        ],
    )
    def sc_kernel(bytes_hbm, tgt_hbm, emb_hbm, wb_hbm, out_hbm, part_hbm,
                  table_v, emb_v, wb_v, idx_v, tgt_v, rows_v, acc_v,
                  in_sem0, in_sem1, out_sem0, out_sem1):
        wid = lax.axis_index("s") * nc + lax.axis_index("c")
        b0 = wid * bw
        in_sems = (in_sem0, in_sem1)
        out_sems = (out_sem0, out_sem1)

        pltpu.sync_copy(emb_hbm, emb_v)
        pltpu.sync_copy(wb_hbm, wb_v)
        iota = lax.iota(jnp.int32, 16)

        def splat(idx):
            return plsc.load_gather(wb_v, [jnp.full((16,), idx, jnp.int32)])

        w = [[splat(4 * k + m) for m in range(4)] for k in range(4)]
        bias = [splat(16 + k) for k in range(4)]

        def table_chunk(t, _):
            e0 = t * 16
            em = [emb_v[m, pl.ds(e0, 16)] for m in range(4)]
            q = []
            for k in range(4):
                qk = bias[k]
                for m in range(4):
                    qk = qk + em[m] * w[k][m]
                q.append(qk)
            n2 = q[0] * q[0] + q[1] * q[1] + q[2] * q[2] + q[3] * q[3]
            inv = _rsqrt16(n2)
            fl = (e0 + iota) * 4
            for k in range(4):
                plsc.store_scatter(table_v, [fl + k], q[k] * inv)
            return _

        lax.fori_loop(0, 16, table_chunk, jnp.float32(0.0))

        def in_copies(st, p):
            s0 = st * 8
            return (
                pltpu.make_async_copy(
                    bytes_hbm.at[pl.ds(s0, 8), pl.ds(b0, bw)],
                    idx_v.at[p], in_sems[p]),
                pltpu.make_async_copy(
                    tgt_hbm.at[pl.ds(s0, 8), :, pl.ds(b0, bw)],
                    tgt_v.at[p], in_sems[p]),
            )

        def out_copy(st, p):
            s0 = st * 8
            return pltpu.make_async_copy(
                rows_v.at[p], out_hbm.at[pl.ds(s0, 8), :, pl.ds(b0, bw)],
                out_sems[p])

        def issue_in(st, p):
            for cp in in_copies(st, p):
                cp.start()

        def wait_in(st, p):
            for cp in in_copies(st, p):
                cp.wait()

        def compute(st, p, acc):
            @plsc.parallel_loop(0, groups, carry=acc, unroll=unroll)
            def body(g, acc):
                r = g // cpr
                c = (g % cpr) * 16
                b16 = idx_v[p, r, pl.ds(c, 16)]
                b4 = b16 * 4
                d = None
                for k in range(4):
                    ck = plsc.load_gather(table_v, [b4 + k])
                    tk = tgt_v[p, r, k, pl.ds(c, 16)]
                    rows_v[p, r, k, pl.ds(c, 16)] = ck
                    d = ck * tk if d is None else d + ck * tk
                return acc + d * d

            return body

        issue_in(0, 0)

        def pair_body(i, acc):
            st0 = 2 * i
            issue_in(st0 + 1, 1)
            wait_in(st0, 0)

            @pl.when(i >= 1)
            def _():
                out_copy(st0, 0).wait()

            acc = compute(st0, 0, acc)
            out_copy(st0, 0).start()
            issue_in(st0 + 2, 0)
            wait_in(st0 + 1, 1)

            @pl.when(i >= 1)
            def _():
                out_copy(st0 + 1, 1).wait()

            acc = compute(st0 + 1, 1, acc)
            out_copy(st0 + 1, 1).start()
            return acc

        acc = lax.fori_loop(0, (n_st - 1) // 2,
                            pair_body, jnp.zeros((16,), jnp.float32))
        last = n_st - 1
        wait_in(last, 0)
        out_copy(last, 0).wait()
        acc = compute(last, 0, acc)
        out_copy(last, 0).start()
        out_copy(last - 1, 1).wait()
        out_copy(last, 0).wait()
        acc_v[...] = acc
        pltpu.sync_copy(acc_v, part_hbm.at[wid])

    return sc_kernel


def kernel(byte_stream, target_geodesic, emb_table, proj_W, proj_b):
    b, s = byte_stream.shape
    n_total = b * s
    bytes_t = byte_stream.T
    tgt_t = jnp.transpose(target_geodesic, (1, 2, 0))
    emb_t = emb_table.T
    wb = jnp.concatenate(
        [proj_W.reshape(16), proj_b,
         jnp.zeros((4,), jnp.float32)]).astype(jnp.float32)

    sc = _make_sc_kernel(s, b)
    out_t, partials = sc(bytes_t, tgt_t, emb_t, wb)

    loss2d = pl.pallas_call(
        functools.partial(_loss_body, float(n_total)),
        out_shape=jax.ShapeDtypeStruct((1, 1), jnp.float32),
    )(partials)
    loss = loss2d.reshape(())

    code_path = jnp.transpose(out_t, (2, 0, 1))
    return (loss, code_path, target_geodesic, jnp.float32(0.1))
